# Initial kernel scaffold; baseline (speedup 1.0000x reference)
import jax
import jax.numpy as jnp
import numpy as np
from jax import lax
from jax.experimental import pallas as pl
from jax.experimental.pallas import tpu as pltpu
from jax.experimental.pallas import tpu_sc as plsc

_N = 10000
_D = 128
_E = 320000

_NC = 2
_NS = 16
_NT = _NC * _NS
_K = 128
_CHUNKS = -(-_E // (_NT * _K))
_EPAD = _NT * _CHUNKS * _K
_RPT = _N // _NS


def _seg_body(h_hbm, src_hbm, dst_hbm, out_hbm, src_v, dst_v, rows_v, acc, sem):
    c = lax.axis_index("c")
    s = lax.axis_index("s")
    g = c * _NS + s

    pltpu.sync_copy(src_hbm.at[g], src_v)
    pltpu.sync_copy(dst_hbm.at[g], dst_v)

    pltpu.sync_copy(h_hbm.at[pl.ds(s * _RPT, _RPT)], acc.at[pl.ds(s * _RPT, _RPT)])
    plsc.subcore_barrier()

    def chunk(j, carry):
        pltpu.async_copy(h_hbm.at[src_v.at[j]], rows_v, sem).wait()
        pltpu.sync_copy(rows_v, acc.at[dst_v.at[j]], add=True)
        return carry

    lax.fori_loop(0, _CHUNKS, chunk, 0)

    plsc.subcore_barrier()
    pltpu.sync_copy(acc.at[pl.ds(s * _RPT, _RPT)],
                    out_hbm.at[c, pl.ds(s * _RPT, _RPT)])


_seg_call = pl.kernel(
    _seg_body,
    out_type=jax.ShapeDtypeStruct((_NC, _N, _D), jnp.float32),
    mesh=plsc.VectorSubcoreMesh(core_axis_name="c", subcore_axis_name="s"),
    scratch_types=[
        pltpu.VMEM((_CHUNKS, _K), jnp.int32),
        pltpu.VMEM((_CHUNKS, _K), jnp.int32),
        pltpu.VMEM((_K, _D), jnp.float32),
        pltpu.VMEM_SHARED((_N + 8, _D), jnp.float32),
        pltpu.SemaphoreType.DMA,
    ],
)


def _mlp_body(h_ref, p_ref, m_ref, wa_ref, ba_ref, wb_ref, bb_ref, o_ref):
    z = p_ref[0] + p_ref[1] - h_ref[...]
    z = jnp.maximum(jnp.dot(z, wa_ref[...], preferred_element_type=jnp.float32)
                    + ba_ref[...], 0.0)
    z = jnp.maximum(jnp.dot(z, wb_ref[...], preferred_element_type=jnp.float32)
                    + bb_ref[...], 0.0)
    o_ref[...] = z * m_ref[...]


def _mlp_final_body(h_ref, p_ref, m_ref, wa_ref, ba_ref, wb_ref, bb_ref,
                    wl_ref, bl_ref, o_ref):
    z = p_ref[0] + p_ref[1] - h_ref[...]
    z = jnp.maximum(jnp.dot(z, wa_ref[...], preferred_element_type=jnp.float32)
                    + ba_ref[...], 0.0)
    z = jnp.maximum(jnp.dot(z, wb_ref[...], preferred_element_type=jnp.float32)
                    + bb_ref[...], 0.0)
    z = z * m_ref[...]
    o_ref[...] = (jnp.dot(z, wl_ref[...], preferred_element_type=jnp.float32)
                  + bl_ref[...])


_BN = 1000
_GRID = _N // _BN

_row_spec = pl.BlockSpec((_BN, _D), lambda i: (i, 0))
_p_spec = pl.BlockSpec((_NC, _BN, _D), lambda i: (0, i, 0))
_w_spec = pl.BlockSpec((_D, _D), lambda i: (0, 0))
_b_spec = pl.BlockSpec((1, _D), lambda i: (0, 0))

_mlp_call = pl.pallas_call(
    _mlp_body,
    grid=(_GRID,),
    in_specs=[_row_spec, _p_spec, _row_spec, _w_spec, _b_spec, _w_spec, _b_spec],
    out_specs=_row_spec,
    out_shape=jax.ShapeDtypeStruct((_N, _D), jnp.float32),
)

_mlp_final_call = pl.pallas_call(
    _mlp_final_body,
    grid=(_GRID,),
    in_specs=[_row_spec, _p_spec, _row_spec, _w_spec, _b_spec, _w_spec, _b_spec,
              _w_spec, _b_spec],
    out_specs=_row_spec,
    out_shape=jax.ShapeDtypeStruct((_N, _D), jnp.float32),
)


_MASK_CACHE = []


def _get_masks():
    if not _MASK_CACHE:
        cpu = jax.local_devices(backend="cpu")[0]
        with jax.default_device(cpu):
            for i in range(3):
                m = jax.random.bernoulli(jax.random.key(1000 + i), 0.9,
                                         (_N, _D))
                _MASK_CACHE.append(
                    np.where(np.asarray(m), np.float32(1.0 / 0.9),
                             np.float32(0.0)))
    return _MASK_CACHE


def kernel(x, edge_index, W0a, b0a, W0b, b0b, W1a, b1a, W1b, b1b,
           W2a, b2a, W2b, b2b, Wl, bl):
    masks = _get_masks()
    src = edge_index[0]
    dst = edge_index[1]
    pad = _EPAD - _E
    src_p = jnp.concatenate([src, jnp.zeros((pad,), jnp.int32)])
    dst_p = jnp.concatenate([dst, jnp.full((pad,), _N, jnp.int32)])
    src3 = src_p.reshape(_NT, _CHUNKS, _K)
    dst3 = dst_p.reshape(_NT, _CHUNKS, _K)

    h = x
    layers = [(W0a, b0a, W0b, b0b), (W1a, b1a, W1b, b1b), (W2a, b2a, W2b, b2b)]
    for i, (Wa, ba, Wb, bb) in enumerate(layers):
        p = _seg_call(h, src3, dst3)
        m = jnp.asarray(masks[i])
        ba2 = ba.reshape(1, _D)
        bb2 = bb.reshape(1, _D)
        if i < 2:
            h = _mlp_call(h, p, m, Wa, ba2, Wb, bb2)
        else:
            h = _mlp_final_call(h, p, m, Wa, ba2, Wb, bb2, Wl,
                                bl.reshape(1, _D))
    return h


# speedup vs baseline: 4.4028x; 4.4028x over previous
import jax
import jax.numpy as jnp
import numpy as np
from jax import lax
from jax.experimental import pallas as pl
from jax.experimental.pallas import tpu as pltpu
from jax.experimental.pallas import tpu_sc as plsc

_N = 10000
_D = 128
_E = 320000

_NC = 2
_NS = 16
_NT = _NC * _NS
_K = 128
_CHUNKS = -(-_E // (_NT * _K))
_EPAD = _NT * _CHUNKS * _K
_RA = 624
_RLAST_OFF = _RA * (_NS - 1)
_RLAST = _N - _RLAST_OFF


def _seg_body(h_hbm, src_hbm, dst_hbm, out_hbm, src_v, dst_v, rows_v, acc, sem):
    c = lax.axis_index("c")
    s = lax.axis_index("s")
    g = c * _NS + s

    pltpu.sync_copy(src_hbm.at[g], src_v)
    pltpu.sync_copy(dst_hbm.at[g], dst_v)

    @pl.when(s < _NS - 1)
    def _():
        pltpu.sync_copy(h_hbm.at[pl.ds(s * _RA, _RA)],
                        acc.at[pl.ds(s * _RA, _RA)])

    @pl.when(s == _NS - 1)
    def _():
        pltpu.sync_copy(h_hbm.at[pl.ds(_RLAST_OFF, _RLAST)],
                        acc.at[pl.ds(_RLAST_OFF, _RLAST)])

    plsc.subcore_barrier()

    def chunk(j, carry):
        pltpu.async_copy(h_hbm.at[src_v.at[j]], rows_v, sem).wait()
        pltpu.sync_copy(rows_v, acc.at[dst_v.at[j]], add=True)
        return carry

    lax.fori_loop(0, _CHUNKS, chunk, 0)

    plsc.subcore_barrier()

    @pl.when(s < _NS - 1)
    def _():
        pltpu.sync_copy(acc.at[pl.ds(s * _RA, _RA)],
                        out_hbm.at[c, pl.ds(s * _RA, _RA)])

    @pl.when(s == _NS - 1)
    def _():
        pltpu.sync_copy(acc.at[pl.ds(_RLAST_OFF, _RLAST)],
                        out_hbm.at[c, pl.ds(_RLAST_OFF, _RLAST)])


_seg_call = pl.kernel(
    _seg_body,
    out_type=jax.ShapeDtypeStruct((_NC, _N, _D), jnp.float32),
    mesh=plsc.VectorSubcoreMesh(core_axis_name="c", subcore_axis_name="s",
                                num_cores=_NC, num_subcores=_NS),
    scratch_types=[
        pltpu.VMEM((_CHUNKS, _K), jnp.int32),
        pltpu.VMEM((_CHUNKS, _K), jnp.int32),
        pltpu.VMEM((_K, _D), jnp.float32),
        pltpu.VMEM_SHARED((_N + 8, _D), jnp.float32),
        pltpu.SemaphoreType.DMA,
    ],
)


def _mlp_body(h_ref, p_ref, m_ref, wa_ref, ba_ref, wb_ref, bb_ref, o_ref):
    z = p_ref[0] + p_ref[1] - h_ref[...]
    z = jnp.maximum(jnp.dot(z, wa_ref[...], preferred_element_type=jnp.float32)
                    + ba_ref[...], 0.0)
    z = jnp.maximum(jnp.dot(z, wb_ref[...], preferred_element_type=jnp.float32)
                    + bb_ref[...], 0.0)
    o_ref[...] = z * m_ref[...]


def _mlp_final_body(h_ref, p_ref, m_ref, wa_ref, ba_ref, wb_ref, bb_ref,
                    wl_ref, bl_ref, o_ref):
    z = p_ref[0] + p_ref[1] - h_ref[...]
    z = jnp.maximum(jnp.dot(z, wa_ref[...], preferred_element_type=jnp.float32)
                    + ba_ref[...], 0.0)
    z = jnp.maximum(jnp.dot(z, wb_ref[...], preferred_element_type=jnp.float32)
                    + bb_ref[...], 0.0)
    z = z * m_ref[...]
    o_ref[...] = (jnp.dot(z, wl_ref[...], preferred_element_type=jnp.float32)
                  + bl_ref[...])


_BN = 1000
_GRID = _N // _BN

_row_spec = pl.BlockSpec((_BN, _D), lambda i: (i, 0))
_p_spec = pl.BlockSpec((_NC, _BN, _D), lambda i: (0, i, 0))
_w_spec = pl.BlockSpec((_D, _D), lambda i: (0, 0))
_b_spec = pl.BlockSpec((1, _D), lambda i: (0, 0))

_mlp_call = pl.pallas_call(
    _mlp_body,
    grid=(_GRID,),
    in_specs=[_row_spec, _p_spec, _row_spec, _w_spec, _b_spec, _w_spec, _b_spec],
    out_specs=_row_spec,
    out_shape=jax.ShapeDtypeStruct((_N, _D), jnp.float32),
)

_mlp_final_call = pl.pallas_call(
    _mlp_final_body,
    grid=(_GRID,),
    in_specs=[_row_spec, _p_spec, _row_spec, _w_spec, _b_spec, _w_spec, _b_spec,
              _w_spec, _b_spec],
    out_specs=_row_spec,
    out_shape=jax.ShapeDtypeStruct((_N, _D), jnp.float32),
)


def _get_masks():
    out = []
    for i in range(3):
        m = jax.random.bernoulli(jax.random.key(1000 + i), 0.9, (_N, _D))
        out.append(jnp.where(m, jnp.float32(1.0 / 0.9), jnp.float32(0.0)))
    return out


def kernel(x, edge_index, W0a, b0a, W0b, b0b, W1a, b1a, W1b, b1b,
           W2a, b2a, W2b, b2b, Wl, bl):
    masks = _get_masks()
    src = edge_index[0]
    dst = edge_index[1]
    pad = _EPAD - _E
    src_p = jnp.concatenate([src, jnp.zeros((pad,), jnp.int32)])
    dst_p = jnp.concatenate([dst, jnp.full((pad,), _N, jnp.int32)])
    src3 = src_p.reshape(_NT, _CHUNKS, _K)
    dst3 = dst_p.reshape(_NT, _CHUNKS, _K)

    h = x
    layers = [(W0a, b0a, W0b, b0b), (W1a, b1a, W1b, b1b), (W2a, b2a, W2b, b2b)]
    for i, (Wa, ba, Wb, bb) in enumerate(layers):
        p = _seg_call(h, src3, dst3)
        m = jnp.asarray(masks[i])
        ba2 = ba.reshape(1, _D)
        bb2 = bb.reshape(1, _D)
        if i < 2:
            h = _mlp_call(h, p, m, Wa, ba2, Wb, bb2)
        else:
            h = _mlp_final_call(h, p, m, Wa, ba2, Wb, bb2, Wl,
                                bl.reshape(1, _D))
    return h
